# Initial kernel scaffold; baseline (speedup 1.0000x reference)
import jax
import jax.numpy as jnp
from jax.experimental import pallas as pl


def kernel(x, edge_index, W, b):
    raise NotImplementedError("write your pallas kernel here")


# speedup vs baseline: 15.6607x; 15.6607x over previous
import functools

import jax
import jax.numpy as jnp
from jax import lax
from jax.experimental import pallas as pl
from jax.experimental.pallas import tpu as pltpu
from jax.experimental.pallas import tpu_sc as plsc

NC = 2
NS = 16
NW = NC * NS
LANES = 16


@functools.lru_cache(maxsize=None)
def _make_deg_kernel(E, NPAD):
    ET = E // NW
    mesh = plsc.VectorSubcoreMesh(core_axis_name="c", subcore_axis_name="s")

    @functools.partial(
        pl.kernel,
        out_type=jax.ShapeDtypeStruct((NW, NPAD), jnp.float32),
        mesh=mesh,
        compiler_params=pltpu.CompilerParams(needs_layout_passes=False),
        scratch_types=[
            pltpu.VMEM((ET,), jnp.int32),
            pltpu.VMEM((NPAD,), jnp.float32),
        ],
    )
    def deg_kernel(dst_hbm, out_hbm, dst_v, deg_v):
        cid = lax.axis_index("c")
        sid = lax.axis_index("s")
        wid = cid * NS + sid

        zero = jnp.zeros((LANES,), jnp.float32)

        def zbody(i, carry):
            deg_v[pl.ds(i * LANES, LANES)] = zero
            return carry

        lax.fori_loop(0, NPAD // LANES, zbody, 0)

        pltpu.sync_copy(dst_hbm.at[pl.ds(wid * ET, ET)], dst_v)

        ones = jnp.ones((LANES,), jnp.float32)

        def body(i, carry):
            idx = dst_v[pl.ds(i * LANES, LANES)]
            plsc.addupdate_scatter(deg_v, [idx], ones)
            return carry

        lax.fori_loop(0, ET // LANES, body, 0)

        pltpu.sync_copy(deg_v, out_hbm.at[wid])

    return deg_kernel


@functools.lru_cache(maxsize=None)
def _make_prep_kernel(N, NPAD, D):
    def prep_kernel(degp_ref, h_ref, dinv_ref, hs_ref):
        degp = degp_ref[...]
        ones = jnp.ones((NW, 1), jnp.float32)
        deg = lax.dot_general(degp, ones, (((0,), (0,)), ((), ())),
                              preferred_element_type=jnp.float32)
        dinv = jnp.where(deg > 0.0,
                         1.0 / jnp.sqrt(jnp.maximum(deg, 1e-12)), 0.0)
        dinv_ref[...] = dinv
        hs_ref[...] = h_ref[...] * dinv[:N]

    return pl.pallas_call(
        prep_kernel,
        out_shape=(
            jax.ShapeDtypeStruct((NPAD, 1), jnp.float32),
            jax.ShapeDtypeStruct((N, D), jnp.float32),
        ),
    )


@functools.lru_cache(maxsize=None)
def _make_msg_kernel(N, E, NPAD, D):
    ET = E // NW
    C = 128
    NFULL = ET // C
    TAIL = ET - NFULL * C
    RPT = NPAD // NS
    mesh = plsc.VectorSubcoreMesh(core_axis_name="c", subcore_axis_name="s")

    scratch = [
        pltpu.VMEM((C,), jnp.int32),
        pltpu.VMEM((C,), jnp.int32),
        pltpu.VMEM((C, D), jnp.float32),
        pltpu.VMEM_SHARED((NPAD, D), jnp.float32),
        pltpu.SemaphoreType.DMA,
    ]
    if TAIL:
        scratch += [
            pltpu.VMEM((TAIL,), jnp.int32),
            pltpu.VMEM((TAIL,), jnp.int32),
            pltpu.VMEM((TAIL, D), jnp.float32),
        ]

    @functools.partial(
        pl.kernel,
        out_type=jax.ShapeDtypeStruct((NC, NPAD, D), jnp.float32),
        mesh=mesh,
        compiler_params=pltpu.CompilerParams(needs_layout_passes=False),
        scratch_types=scratch,
    )
    def msg_kernel(hs_hbm, src_hbm, dst_hbm, zeros_hbm, out_hbm,
                   src_v, dst_v, rows_v, acc, sem, *tail_bufs):
        cid = lax.axis_index("c")
        sid = lax.axis_index("s")
        wid = cid * NS + sid

        pltpu.sync_copy(zeros_hbm.at[pl.ds(sid * RPT, RPT)],
                        acc.at[pl.ds(sid * RPT, RPT)])
        plsc.subcore_barrier()

        base = wid * ET

        def body(i, carry):
            off = base + i * C
            pltpu.sync_copy(src_hbm.at[pl.ds(off, C)], src_v)
            pltpu.sync_copy(dst_hbm.at[pl.ds(off, C)], dst_v)
            pltpu.async_copy(hs_hbm.at[src_v], rows_v, sem).wait()
            pltpu.sync_copy(rows_v, acc.at[dst_v], add=True)
            return carry

        lax.fori_loop(0, NFULL, body, 0)

        if TAIL:
            srct_v, dstt_v, rowst_v = tail_bufs
            off = base + NFULL * C
            pltpu.sync_copy(src_hbm.at[pl.ds(off, TAIL)], srct_v)
            pltpu.sync_copy(dst_hbm.at[pl.ds(off, TAIL)], dstt_v)
            pltpu.async_copy(hs_hbm.at[srct_v], rowst_v, sem).wait()
            pltpu.sync_copy(rowst_v, acc.at[dstt_v], add=True)

        plsc.subcore_barrier()
        pltpu.sync_copy(acc.at[pl.ds(sid * RPT, RPT)],
                        out_hbm.at[cid].at[pl.ds(sid * RPT, RPT)])

    return msg_kernel


@functools.lru_cache(maxsize=None)
def _make_gram_kernel(N, NPAD, D, DO, BM, BN):
    nI = (N + BM - 1) // BM
    nJ = (N + BN - 1) // BN

    def gram_kernel(aggp_ref, dinv_ref, w_ref, b_ref, out_ref, hp_ref):
        i = pl.program_id(0)
        j = pl.program_id(1)

        @pl.when((i == 0) & (j == 0))
        def _():
            agg = (aggp_ref[0] + aggp_ref[1]) * dinv_ref[...]
            hp = jnp.dot(agg, w_ref[...],
                         preferred_element_type=jnp.float32) + b_ref[...]
            hp_ref[...] = jnp.maximum(hp, 0.0)

        hi = hp_ref[pl.ds(i * BM, BM), :]
        hj = hp_ref[pl.ds(j * BN, BN), :]
        out_ref[...] = lax.dot_general(hi, hj, (((1,), (1,)), ((), ())),
                                       preferred_element_type=jnp.float32)

    return pl.pallas_call(
        gram_kernel,
        grid=(nI, nJ),
        in_specs=[
            pl.BlockSpec((NC, NPAD, D), lambda i, j: (0, 0, 0)),
            pl.BlockSpec((NPAD, 1), lambda i, j: (0, 0)),
            pl.BlockSpec((D, DO), lambda i, j: (0, 0)),
            pl.BlockSpec((1, DO), lambda i, j: (0, 0)),
        ],
        out_specs=pl.BlockSpec((BM, BN), lambda i, j: (i, j)),
        out_shape=jax.ShapeDtypeStruct((N, N), jnp.float32),
        scratch_shapes=[pltpu.VMEM((NPAD, DO), jnp.float32)],
    )


def kernel(x, edge_index, W, b):
    N, D = x.shape[1], x.shape[2]
    DO = W.shape[1]
    E = edge_index.shape[1]
    NPAD = -(-N // 1024) * 1024

    h = x[0]
    src = edge_index[0]
    dst = edge_index[1]

    deg_part = _make_deg_kernel(E, NPAD)(dst)
    dinv, hs = _make_prep_kernel(N, NPAD, D)(deg_part, h)
    zeros = jnp.zeros((NPAD, D), jnp.float32)
    agg_part = _make_msg_kernel(N, E, NPAD, D)(hs, src, dst, zeros)
    out = _make_gram_kernel(N, NPAD, D, DO, 1024, 1024)(
        agg_part, dinv, W, b.reshape(1, DO))
    return out
